# Initial kernel scaffold; baseline (speedup 1.0000x reference)
import jax
import jax.numpy as jnp
from jax.experimental import pallas as pl


def kernel(center, target, in_embeddings, inner_vectors, paths, codes, masks):
    raise NotImplementedError("write your pallas kernel here")


# speedup vs baseline: 25.2938x; 25.2938x over previous
import functools

import jax
import jax.numpy as jnp
from jax import lax
from jax.experimental import pallas as pl
from jax.experimental.pallas import tpu as pltpu
from jax.experimental.pallas import tpu_sc as plsc

VOCAB = 64
EMBED = 128
DEPTH = 6
BATCH = 16384

_NUM_WORKERS = 32
_CHUNK = BATCH // _NUM_WORKERS
_LANES = 16
_STEPS = _CHUNK // _LANES
_PAIRS = VOCAB * VOCAB


def _table_body(e_ref, w_ref, paths_ref, codes_ref, masks_ref, out_ref):
    scores = lax.dot_general(
        e_ref[...], w_ref[...], (((1,), (1,)), ((), ())),
        preferred_element_type=jnp.float32,
    )
    acc = jnp.zeros((VOCAB, VOCAB), jnp.float32)
    node_iota = lax.broadcasted_iota(jnp.int32, (VOCAB, VOCAB), 1)
    for l in range(DEPTH):
        p_col = paths_ref[:, l : l + 1]
        onehot = (p_col == node_iota).astype(jnp.float32)
        g = lax.dot_general(
            onehot, scores, (((1,), (1,)), ((), ())),
            preferred_element_type=jnp.float32,
        )
        x = codes_ref[:, l : l + 1] * g
        ls = jnp.minimum(x, 0.0) - jnp.log(1.0 + jnp.exp(-jnp.abs(x)))
        acc = acc + ls * masks_ref[:, l : l + 1]
    out_ref[...] = -acc


def _gather_body(table_hbm, center_hbm, target_hbm, out_hbm,
                 tab_v, c_v, t_v, part_v):
    wid = lax.axis_index("s") * 2 + lax.axis_index("c")
    base = wid * _CHUNK
    pltpu.sync_copy(table_hbm, tab_v)
    pltpu.sync_copy(center_hbm.at[pl.ds(base, _CHUNK)], c_v)
    pltpu.sync_copy(target_hbm.at[pl.ds(base, _CHUNK)], t_v)

    acc = jnp.zeros((_LANES,), jnp.float32)
    for i in range(_STEPS):
        c = c_v[pl.ds(i * _LANES, _LANES)]
        t = t_v[pl.ds(i * _LANES, _LANES)]
        pair = t * VOCAB + c
        acc = acc + plsc.load_gather(tab_v, [pair])
    part_v[...] = acc
    pltpu.sync_copy(part_v, out_hbm.at[wid])


def _final_body(p_ref, o_ref):
    o_ref[0, 0] = jnp.sum(p_ref[...]) * (1.0 / BATCH)


@functools.lru_cache(maxsize=None)
def _gather_sum():
    return pl.kernel(
        _gather_body,
        out_type=jax.ShapeDtypeStruct((_NUM_WORKERS, _LANES), jnp.float32),
        mesh=plsc.VectorSubcoreMesh(core_axis_name="c", subcore_axis_name="s"),
        compiler_params=pltpu.CompilerParams(needs_layout_passes=False),
        scratch_types=[
            pltpu.VMEM((_PAIRS,), jnp.float32),
            pltpu.VMEM((_CHUNK,), jnp.int32),
            pltpu.VMEM((_CHUNK,), jnp.int32),
            pltpu.VMEM((_LANES,), jnp.float32),
        ],
    )


def kernel(center, target, in_embeddings, inner_vectors, paths, codes, masks):
    w_pad = jnp.concatenate(
        [inner_vectors, jnp.zeros((1, EMBED), inner_vectors.dtype)], axis=0)
    table = pl.pallas_call(
        _table_body,
        out_shape=jax.ShapeDtypeStruct((VOCAB, VOCAB), jnp.float32),
    )(in_embeddings, w_pad, paths.astype(jnp.int32),
      codes.astype(jnp.float32), masks.astype(jnp.float32))

    partials = _gather_sum()(
        table.reshape(_PAIRS), center.astype(jnp.int32),
        target.astype(jnp.int32))

    total = pl.pallas_call(
        _final_body,
        out_shape=jax.ShapeDtypeStruct((1, 1), jnp.float32),
        out_specs=pl.BlockSpec(memory_space=pltpu.SMEM),
    )(partials)
    return total[0, 0]


# speedup vs baseline: 26.5236x; 1.0486x over previous
import functools

import jax
import jax.numpy as jnp
from jax import lax
from jax.experimental import pallas as pl
from jax.experimental.pallas import tpu as pltpu
from jax.experimental.pallas import tpu_sc as plsc

VOCAB = 64
EMBED = 128
DEPTH = 6
BATCH = 16384

_NUM_WORKERS = 32
_CHUNK = BATCH // _NUM_WORKERS
_LANES = 16
_STEPS = _CHUNK // _LANES
_PAIRS = VOCAB * VOCAB


def _hist_body(center_hbm, target_hbm, out_hbm, hist_v, c_v, t_v):
    wid = lax.axis_index("s") * 2 + lax.axis_index("c")
    base = wid * _CHUNK
    pltpu.sync_copy(center_hbm.at[pl.ds(base, _CHUNK)], c_v)
    pltpu.sync_copy(target_hbm.at[pl.ds(base, _CHUNK)], t_v)

    zeros = jnp.zeros((_LANES,), jnp.float32)
    for i in range(_PAIRS // _LANES):
        hist_v[pl.ds(i * _LANES, _LANES)] = zeros

    ones = jnp.ones((_LANES,), jnp.float32)
    for i in range(_STEPS):
        c = c_v[pl.ds(i * _LANES, _LANES)]
        t = t_v[pl.ds(i * _LANES, _LANES)]
        pair = t * VOCAB + c
        plsc.addupdate_scatter(hist_v, [pair], ones)

    pltpu.sync_copy(hist_v, out_hbm.at[wid])


@functools.lru_cache(maxsize=None)
def _hist_kernel():
    return pl.kernel(
        _hist_body,
        out_type=jax.ShapeDtypeStruct((_NUM_WORKERS, _PAIRS), jnp.float32),
        mesh=plsc.VectorSubcoreMesh(core_axis_name="c", subcore_axis_name="s"),
        compiler_params=pltpu.CompilerParams(needs_layout_passes=False),
        scratch_types=[
            pltpu.VMEM((_PAIRS,), jnp.float32),
            pltpu.VMEM((_CHUNK,), jnp.int32),
            pltpu.VMEM((_CHUNK,), jnp.int32),
        ],
    )


def _table_body(cnt_ref, e_ref, w_ref, paths_ref, codes_ref, masks_ref, o_ref):
    counts = jnp.sum(cnt_ref[...], axis=0)
    scores = lax.dot_general(
        e_ref[...], w_ref[...], (((1,), (1,)), ((), ())),
        preferred_element_type=jnp.float32,
    )
    acc = jnp.zeros((VOCAB, VOCAB), jnp.float32)
    node_iota = lax.broadcasted_iota(jnp.int32, (VOCAB, VOCAB), 1)
    for l in range(DEPTH):
        p_col = paths_ref[:, l : l + 1]
        onehot = (p_col == node_iota).astype(jnp.float32)
        g = lax.dot_general(
            onehot, scores, (((1,), (1,)), ((), ())),
            preferred_element_type=jnp.float32,
        )
        x = codes_ref[:, l : l + 1] * g
        ls = jnp.minimum(x, 0.0) - jnp.log(1.0 + jnp.exp(-jnp.abs(x)))
        acc = acc + ls * masks_ref[:, l : l + 1]
    o_ref[0, 0] = jnp.sum(counts * (-acc)) * (1.0 / BATCH)


def kernel(center, target, in_embeddings, inner_vectors, paths, codes, masks):
    hist = _hist_kernel()(center.astype(jnp.int32), target.astype(jnp.int32))

    w_pad = jnp.concatenate(
        [inner_vectors, jnp.zeros((1, EMBED), inner_vectors.dtype)], axis=0)
    total = pl.pallas_call(
        _table_body,
        out_shape=jax.ShapeDtypeStruct((1, 1), jnp.float32),
        out_specs=pl.BlockSpec(memory_space=pltpu.SMEM),
    )(hist.reshape(_NUM_WORKERS, VOCAB, VOCAB), in_embeddings, w_pad,
      paths.astype(jnp.int32), codes.astype(jnp.float32),
      masks.astype(jnp.float32))
    return total[0, 0]


# speedup vs baseline: 28.9173x; 1.0902x over previous
import functools

import jax
import jax.numpy as jnp
from jax import lax
from jax.experimental import pallas as pl
from jax.experimental.pallas import tpu as pltpu
from jax.experimental.pallas import tpu_sc as plsc

VOCAB = 64
EMBED = 128
DEPTH = 6
BATCH = 16384

_NUM_WORKERS = 32
_CHUNK = BATCH // _NUM_WORKERS
_LANES = 16
_STEPS = _CHUNK // _LANES
_PAIRS = VOCAB * VOCAB


def _hist_body(center_hbm, target_hbm, out_hbm, hist_v, c_v, t_v):
    wid = lax.axis_index("s") * 2 + lax.axis_index("c")
    base = wid * _CHUNK
    pltpu.sync_copy(center_hbm.at[pl.ds(base, _CHUNK)], c_v)
    pltpu.sync_copy(target_hbm.at[pl.ds(base, _CHUNK)], t_v)

    zeros = jnp.zeros((_LANES,), jnp.float32)
    for r in range(VOCAB):
        for j in range(VOCAB // _LANES):
            hist_v[r, pl.ds(j * _LANES, _LANES)] = zeros

    ones = jnp.ones((_LANES,), jnp.float32)
    for i in range(_STEPS):
        c = c_v[pl.ds(i * _LANES, _LANES)]
        t = t_v[pl.ds(i * _LANES, _LANES)]
        plsc.addupdate_scatter(hist_v, [t, c], ones)

    pltpu.sync_copy(hist_v, out_hbm.at[wid])


@functools.lru_cache(maxsize=None)
def _hist_kernel():
    return pl.kernel(
        _hist_body,
        out_type=jax.ShapeDtypeStruct((_NUM_WORKERS, VOCAB, VOCAB),
                                      jnp.float32),
        mesh=plsc.VectorSubcoreMesh(core_axis_name="c", subcore_axis_name="s"),
        compiler_params=pltpu.CompilerParams(needs_layout_passes=False),
        scratch_types=[
            pltpu.VMEM((VOCAB, VOCAB), jnp.float32),
            pltpu.VMEM((_CHUNK,), jnp.int32),
            pltpu.VMEM((_CHUNK,), jnp.int32),
        ],
    )


def _table_body(cnt_ref, e_ref, w_ref, paths_ref, codes_ref, masks_ref, o_ref):
    counts = jnp.sum(cnt_ref[...], axis=0)
    scores = lax.dot_general(
        e_ref[...], w_ref[...], (((1,), (1,)), ((), ())),
        preferred_element_type=jnp.float32,
    )
    acc = jnp.zeros((VOCAB, VOCAB), jnp.float32)
    node_iota = lax.broadcasted_iota(jnp.int32, (VOCAB, VOCAB - 1), 1)
    for l in range(DEPTH):
        p_col = paths_ref[:, l : l + 1]
        onehot = (p_col == node_iota).astype(jnp.float32)
        g = lax.dot_general(
            onehot, scores, (((1,), (1,)), ((), ())),
            preferred_element_type=jnp.float32,
        )
        x = codes_ref[:, l : l + 1] * g
        ls = jnp.minimum(x, 0.0) - jnp.log(1.0 + jnp.exp(-jnp.abs(x)))
        acc = acc + ls * masks_ref[:, l : l + 1]
    o_ref[0, 0] = jnp.sum(counts * (-acc)) * (1.0 / BATCH)


def kernel(center, target, in_embeddings, inner_vectors, paths, codes, masks):
    hist = _hist_kernel()(center.astype(jnp.int32), target.astype(jnp.int32))

    total = pl.pallas_call(
        _table_body,
        out_shape=jax.ShapeDtypeStruct((1, 1), jnp.float32),
        out_specs=pl.BlockSpec(memory_space=pltpu.SMEM),
    )(hist, in_embeddings, inner_vectors,
      paths.astype(jnp.int32), codes.astype(jnp.float32),
      masks.astype(jnp.float32))
    return total[0, 0]


# speedup vs baseline: 30.0941x; 1.0407x over previous
import functools

import jax
import jax.numpy as jnp
from jax import lax
from jax.experimental import pallas as pl
from jax.experimental.pallas import tpu as pltpu
from jax.experimental.pallas import tpu_sc as plsc

VOCAB = 64
EMBED = 128
DEPTH = 6
BATCH = 16384

_NUM_WORKERS = 32
_CHUNK = BATCH // _NUM_WORKERS
_LANES = 16
_STEPS = _CHUNK // _LANES
_PAIRS = VOCAB * VOCAB


def _hist_body(center_hbm, target_hbm, out_hbm, hist_v, c_v, t_v):
    wid = lax.axis_index("s") * 2 + lax.axis_index("c")
    base = wid * _CHUNK
    pltpu.sync_copy(center_hbm.at[pl.ds(base, _CHUNK)], c_v)
    pltpu.sync_copy(target_hbm.at[pl.ds(base, _CHUNK)], t_v)

    zeros = jnp.zeros((_LANES,), jnp.float32)

    def zero_row(r, carry):
        for j in range(VOCAB // _LANES):
            hist_v[r, pl.ds(j * _LANES, _LANES)] = zeros
        return carry

    lax.fori_loop(0, VOCAB, zero_row, 0)

    ones = jnp.ones((_LANES,), jnp.float32)

    def step(i, carry):
        c = c_v[pl.ds(i * _LANES, _LANES)]
        t = t_v[pl.ds(i * _LANES, _LANES)]
        plsc.addupdate_scatter(hist_v, [t, c], ones)
        return carry

    lax.fori_loop(0, _STEPS, step, 0)

    pltpu.sync_copy(hist_v, out_hbm.at[wid])


@functools.lru_cache(maxsize=None)
def _hist_kernel():
    return pl.kernel(
        _hist_body,
        out_type=jax.ShapeDtypeStruct((_NUM_WORKERS, VOCAB, VOCAB),
                                      jnp.float32),
        mesh=plsc.VectorSubcoreMesh(core_axis_name="c", subcore_axis_name="s"),
        compiler_params=pltpu.CompilerParams(needs_layout_passes=False),
        scratch_types=[
            pltpu.VMEM((VOCAB, VOCAB), jnp.float32),
            pltpu.VMEM((_CHUNK,), jnp.int32),
            pltpu.VMEM((_CHUNK,), jnp.int32),
        ],
    )


def _table_body(cnt_ref, e_ref, w_ref, paths_ref, codes_ref, masks_ref, o_ref):
    counts = jnp.sum(cnt_ref[...], axis=0)
    scores = lax.dot_general(
        e_ref[...], w_ref[...], (((1,), (1,)), ((), ())),
        preferred_element_type=jnp.float32,
    )
    node_iota = lax.broadcasted_iota(jnp.int32, (VOCAB, VOCAB - 1), 1)
    onehot = jnp.concatenate(
        [(paths_ref[:, l : l + 1] == node_iota).astype(jnp.float32)
         for l in range(DEPTH)], axis=0)
    g = lax.dot_general(
        onehot, scores, (((1,), (1,)), ((), ())),
        preferred_element_type=jnp.float32,
    )
    code_col = jnp.concatenate(
        [codes_ref[:, l : l + 1] for l in range(DEPTH)], axis=0)
    mask_col = jnp.concatenate(
        [masks_ref[:, l : l + 1] for l in range(DEPTH)], axis=0)
    cnt_rep = jnp.concatenate([counts] * DEPTH, axis=0)
    x = code_col * g
    ls = jnp.minimum(x, 0.0) - jnp.log(1.0 + jnp.exp(-jnp.abs(x)))
    o_ref[0, 0] = jnp.sum(cnt_rep * (ls * mask_col)) * (-1.0 / BATCH)


def kernel(center, target, in_embeddings, inner_vectors, paths, codes, masks):
    hist = _hist_kernel()(center.astype(jnp.int32), target.astype(jnp.int32))

    total = pl.pallas_call(
        _table_body,
        out_shape=jax.ShapeDtypeStruct((1, 1), jnp.float32),
        out_specs=pl.BlockSpec(memory_space=pltpu.SMEM),
    )(hist, in_embeddings, inner_vectors,
      paths.astype(jnp.int32), codes.astype(jnp.float32),
      masks.astype(jnp.float32))
    return total[0, 0]


# speedup vs baseline: 31.2910x; 1.0398x over previous
import functools

import jax
import jax.numpy as jnp
from jax import lax
from jax.experimental import pallas as pl
from jax.experimental.pallas import tpu as pltpu
from jax.experimental.pallas import tpu_sc as plsc

VOCAB = 64
EMBED = 128
DEPTH = 6
BATCH = 16384

_NUM_WORKERS = 32
_CHUNK = BATCH // _NUM_WORKERS
_LANES = 16
_STEPS = _CHUNK // _LANES
_PAIRS = VOCAB * VOCAB


def _hist_body(center_hbm, target_hbm, out_hbm, hist_v, c_v, t_v, sem):
    wid = lax.axis_index("s") * 2 + lax.axis_index("c")
    base = wid * _CHUNK
    cp_c = pltpu.async_copy(center_hbm.at[pl.ds(base, _CHUNK)], c_v, sem)
    cp_t = pltpu.async_copy(target_hbm.at[pl.ds(base, _CHUNK)], t_v, sem)

    zeros = jnp.zeros((_LANES,), jnp.float32)

    def zero_row(r, carry):
        for j in range(VOCAB // _LANES):
            hist_v[r, pl.ds(j * _LANES, _LANES)] = zeros
        return carry

    lax.fori_loop(0, VOCAB, zero_row, 0)
    cp_c.wait()
    cp_t.wait()

    ones = jnp.ones((_LANES,), jnp.float32)

    def step(i, carry):
        c = c_v[pl.ds(i * _LANES, _LANES)]
        t = t_v[pl.ds(i * _LANES, _LANES)]
        plsc.addupdate_scatter(hist_v, [t, c], ones)
        return carry

    lax.fori_loop(0, _STEPS, step, 0)

    pltpu.sync_copy(hist_v, out_hbm.at[wid])


@functools.lru_cache(maxsize=None)
def _hist_kernel():
    return pl.kernel(
        _hist_body,
        out_type=jax.ShapeDtypeStruct((_NUM_WORKERS, VOCAB, VOCAB),
                                      jnp.float32),
        mesh=plsc.VectorSubcoreMesh(core_axis_name="c", subcore_axis_name="s"),
        compiler_params=pltpu.CompilerParams(needs_layout_passes=False),
        scratch_types=[
            pltpu.VMEM((VOCAB, VOCAB), jnp.float32),
            pltpu.VMEM((_CHUNK,), jnp.int32),
            pltpu.VMEM((_CHUNK,), jnp.int32),
            pltpu.SemaphoreType.DMA,
        ],
    )


def _table_body(e_ref, w_ref, paths_ref, codes_ref, masks_ref, o_ref):
    scores = lax.dot_general(
        e_ref[...], w_ref[...], (((1,), (1,)), ((), ())),
        preferred_element_type=jnp.float32,
    )
    node_iota = lax.broadcasted_iota(jnp.int32, (VOCAB, VOCAB - 1), 1)
    onehot = jnp.concatenate(
        [(paths_ref[:, l : l + 1] == node_iota).astype(jnp.float32)
         for l in range(DEPTH)], axis=0)
    g = lax.dot_general(
        onehot, scores, (((1,), (1,)), ((), ())),
        preferred_element_type=jnp.float32,
    )
    code_col = jnp.concatenate(
        [codes_ref[:, l : l + 1] for l in range(DEPTH)], axis=0)
    mask_col = jnp.concatenate(
        [masks_ref[:, l : l + 1] for l in range(DEPTH)], axis=0)
    x = code_col * g
    ls = jnp.minimum(x, 0.0) - jnp.log(1.0 + jnp.exp(-jnp.abs(x)))
    wl = ls * mask_col
    table = (wl[0:64] + wl[64:128] + wl[128:192] + wl[192:256]
             + wl[256:320] + wl[320:384])
    o_ref[...] = table * (-1.0 / BATCH)


def _combine_body(cnt_ref, table_ref, o_ref):
    counts = jnp.sum(cnt_ref[...], axis=0)
    o_ref[0, 0] = jnp.sum(counts * table_ref[...])


def kernel(center, target, in_embeddings, inner_vectors, paths, codes, masks):
    center = center.astype(jnp.int32) if center.dtype != jnp.int32 else center
    target = target.astype(jnp.int32) if target.dtype != jnp.int32 else target
    hist = _hist_kernel()(center, target)

    table = pl.pallas_call(
        _table_body,
        out_shape=jax.ShapeDtypeStruct((VOCAB, VOCAB), jnp.float32),
    )(in_embeddings, inner_vectors,
      paths.astype(jnp.int32) if paths.dtype != jnp.int32 else paths,
      codes, masks)

    total = pl.pallas_call(
        _combine_body,
        out_shape=jax.ShapeDtypeStruct((1, 1), jnp.float32),
        out_specs=pl.BlockSpec(memory_space=pltpu.SMEM),
    )(hist, table)
    return total[0, 0]
